# baseline (speedup 1.0000x reference)
import functools

import jax
import jax.numpy as jnp
from jax import lax
from jax.experimental import pallas as pl
from jax.experimental.pallas import tpu as pltpu
from jax.experimental.pallas import tpu_sc as plsc

N = 10000
E = 320000
D = 128
C = 40
CP = 64

NC = 2
NS = 16
NW = NC * NS
CH = 128
STEPS = 80
HALF = STEPS // 2
EPW = STEPS * CH
EPAD = NW * EPW
NB = 2
NPAD = 10240
RPS = NPAD // NS

_mesh = plsc.VectorSubcoreMesh(core_axis_name="c", subcore_axis_name="s")


def _make_segsum(W):
    @functools.partial(
        pl.kernel,
        out_type=jax.ShapeDtypeStruct((NC * NPAD, W), jnp.float32),
        mesh=_mesh,
        compiler_params=pltpu.CompilerParams(use_tc_tiling_on_sc=False),
        scratch_types=[
            pltpu.VMEM((STEPS, CH), jnp.int32),
            pltpu.VMEM((HALF, CH), jnp.int32),
            pltpu.VMEM((NB, CH, W), jnp.float32),
            pltpu.VMEM_SHARED((NPAD, W), jnp.float32),
            pltpu.SemaphoreType.DMA((NB,)),
            pltpu.SemaphoreType.DMA,
        ],
    )
    def _segsum(h_hbm, src_hbm, dst_hbm, z_hbm, out_hbm, sidx, didx, bufs,
                acc, gsem, zsem):
        cid = lax.axis_index("c")
        sid = lax.axis_index("s")
        wid = cid * NS + sid

        zbase = sid * RPS
        pltpu.async_copy(z_hbm, acc.at[pl.ds(zbase, RPS)], zsem)

        pltpu.sync_copy(src_hbm.at[pl.ds(wid * STEPS, STEPS)], sidx)

        for b in range(NB):
            pltpu.async_copy(h_hbm.at[sidx.at[b]], bufs.at[b], gsem.at[b])

        pltpu.make_async_copy(z_hbm, acc.at[pl.ds(zbase, RPS)], zsem).wait()
        plsc.subcore_barrier()

        for h in range(2):
            pltpu.sync_copy(dst_hbm.at[pl.ds(wid * STEPS + h * HALF, HALF)],
                            didx)
            ngroups = HALF // NB if h == 0 else HALF // NB - 1

            def group(g, _, h=h):
                for b in range(NB):
                    i = h * HALF + g * NB + b
                    pltpu.make_async_copy(h_hbm.at[sidx.at[i]], bufs.at[b],
                                          gsem.at[b]).wait()
                    pltpu.sync_copy(bufs.at[b], acc.at[didx.at[g * NB + b]],
                                    add=True)
                    pltpu.async_copy(h_hbm.at[sidx.at[i + NB]], bufs.at[b],
                                     gsem.at[b])
                return 0

            lax.fori_loop(0, ngroups, group, 0)

        for b in range(NB):
            i = STEPS - NB + b
            pltpu.make_async_copy(h_hbm.at[sidx.at[i]], bufs.at[b],
                                  gsem.at[b]).wait()
            pltpu.sync_copy(bufs.at[b], acc.at[didx.at[HALF - NB + b]],
                            add=True)

        plsc.subcore_barrier()

        obase = cid * NPAD + sid * RPS
        pltpu.sync_copy(acc.at[pl.ds(sid * RPS, RPS)],
                        out_hbm.at[pl.ds(obase, RPS)])

    return _segsum


_segsum_d = _make_segsum(D)
_segsum_c = _make_segsum(CP)

BN = 2000


def _update_body(relu, m_ref, norm_ref, w_ref, b_ref, o_ref):
    m = m_ref[0] + m_ref[1]
    m = m * norm_ref[...]
    acc = jnp.dot(m, w_ref[...], preferred_element_type=jnp.float32)
    acc = acc + b_ref[...]
    if relu:
        acc = jnp.maximum(acc, 0.0)
    o_ref[...] = acc


def _update(m2, norm, W, b, relu):
    H = W.shape[1]
    return pl.pallas_call(
        functools.partial(_update_body, relu),
        grid=(N // BN,),
        in_specs=[
            pl.BlockSpec((2, BN, D), lambda i: (0, i, 0)),
            pl.BlockSpec((BN, 1), lambda i: (i, 0)),
            pl.BlockSpec((D, H), lambda i: (0, 0)),
            pl.BlockSpec((1, H), lambda i: (0, 0)),
        ],
        out_specs=pl.BlockSpec((BN, H), lambda i: (i, 0)),
        out_shape=jax.ShapeDtypeStruct((N, H), jnp.float32),
    )(m2, norm, W, b.reshape(1, H))


def _update_proj_body(m_ref, norm_ref, w_ref, b_ref, wp_ref, o_ref):
    m = (m_ref[0] + m_ref[1]) * norm_ref[...]
    hid = jnp.dot(m, w_ref[...], preferred_element_type=jnp.float32)
    hid = jnp.maximum(hid + b_ref[...], 0.0)
    o_ref[...] = jnp.dot(hid, wp_ref[...], preferred_element_type=jnp.float32)


def _update_proj(m2, norm, W, b, Wp):
    return pl.pallas_call(
        _update_proj_body,
        grid=(N // BN,),
        in_specs=[
            pl.BlockSpec((2, BN, D), lambda i: (0, i, 0)),
            pl.BlockSpec((BN, 1), lambda i: (i, 0)),
            pl.BlockSpec((D, D), lambda i: (0, 0)),
            pl.BlockSpec((1, D), lambda i: (0, 0)),
            pl.BlockSpec((D, CP), lambda i: (0, 0)),
        ],
        out_specs=pl.BlockSpec((BN, CP), lambda i: (i, 0)),
        out_shape=jax.ShapeDtypeStruct((N, CP), jnp.float32),
    )(m2, norm, W, b.reshape(1, D), Wp)


def _final_body(m_ref, norm_ref, b_ref, o_ref):
    m = m_ref[0, :, :C] + m_ref[1, :, :C]
    o_ref[...] = m * norm_ref[...] + b_ref[...]


def _final(m2, norm, b):
    return pl.pallas_call(
        _final_body,
        grid=(N // BN,),
        in_specs=[
            pl.BlockSpec((2, BN, CP), lambda i: (0, i, 0)),
            pl.BlockSpec((BN, 1), lambda i: (i, 0)),
            pl.BlockSpec((1, C), lambda i: (0, 0)),
        ],
        out_specs=pl.BlockSpec((BN, C), lambda i: (i, 0)),
        out_shape=jax.ShapeDtypeStruct((N, C), jnp.float32),
    )(m2, norm, b.reshape(1, C))


@jax.jit
def kernel(x, edge_index, norm, W0, b0, W1, b1, W2, b2):
    pad = jnp.arange(EPAD - E, dtype=jnp.int32)
    src = jnp.concatenate([edge_index[0], pad % N]).reshape(-1, CH)
    dst = jnp.concatenate([edge_index[1], N + pad % (NPAD - N)]).reshape(-1, CH)
    z = jnp.zeros((RPS, D), jnp.float32)
    zc = jnp.zeros((RPS, CP), jnp.float32)

    Wp = jnp.pad(W2, ((0, 0), (0, CP - C)))

    m = _segsum_d(x, src, dst, z).reshape(2, NPAD, D)
    h = _update(m, norm, W0, b0, relu=True)
    m = _segsum_d(h, src, dst, z).reshape(2, NPAD, D)
    p = _update_proj(m, norm, W1, b1, Wp)
    m = _segsum_c(p, src, dst, zc).reshape(2, NPAD, CP)
    return _final(m, norm, b2)


# speedup vs baseline: 1.0109x; 1.0109x over previous
import functools

import jax
import jax.numpy as jnp
from jax import lax
from jax.experimental import pallas as pl
from jax.experimental.pallas import tpu as pltpu
from jax.experimental.pallas import tpu_sc as plsc

N = 10000
E = 320000
D = 128
C = 40
CP = 64

NC = 2
NS = 16
NW = NC * NS
CH = 128
STEPS = 80
HALF = STEPS // 2
EPW = STEPS * CH
EPAD = NW * EPW
NB = 2
NPAD = 10240
RPS = NPAD // NS

_mesh = plsc.VectorSubcoreMesh(core_axis_name="c", subcore_axis_name="s")


def _make_segsum(W):
    @functools.partial(
        pl.kernel,
        out_type=jax.ShapeDtypeStruct((NC * NPAD, W), jnp.float32),
        mesh=_mesh,
        compiler_params=pltpu.CompilerParams(use_tc_tiling_on_sc=False),
        scratch_types=[
            pltpu.VMEM((STEPS, CH), jnp.int32),
            pltpu.VMEM((HALF, CH), jnp.int32),
            pltpu.VMEM((NB, CH, W), jnp.float32),
            pltpu.VMEM_SHARED((NPAD, W), jnp.float32),
            pltpu.SemaphoreType.DMA((NB,)),
            pltpu.SemaphoreType.DMA,
        ],
    )
    def _segsum(h_hbm, src_hbm, dst_hbm, z_hbm, out_hbm, sidx, didx, bufs,
                acc, gsem, zsem):
        cid = lax.axis_index("c")
        sid = lax.axis_index("s")
        wid = cid * NS + sid

        zbase = sid * RPS
        pltpu.async_copy(z_hbm.at[pl.ds(zbase, RPS)],
                         acc.at[pl.ds(zbase, RPS)], zsem)

        pltpu.sync_copy(src_hbm.at[pl.ds(wid * STEPS, STEPS)], sidx)

        for b in range(NB):
            pltpu.async_copy(h_hbm.at[sidx.at[b]], bufs.at[b], gsem.at[b])

        pltpu.make_async_copy(z_hbm.at[pl.ds(zbase, RPS)],
                              acc.at[pl.ds(zbase, RPS)], zsem).wait()
        plsc.subcore_barrier()

        for h in range(2):
            pltpu.sync_copy(dst_hbm.at[pl.ds(wid * STEPS + h * HALF, HALF)],
                            didx)
            UNR = 4
            ngroups = HALF // UNR if h == 0 else HALF // UNR - 1

            def group(g, _, h=h):
                for u in range(UNR):
                    b = u % NB
                    i = h * HALF + g * UNR + u
                    pltpu.make_async_copy(h_hbm.at[sidx.at[i]], bufs.at[b],
                                          gsem.at[b]).wait()
                    pltpu.sync_copy(bufs.at[b], acc.at[didx.at[g * UNR + u]],
                                    add=True)
                    pltpu.async_copy(h_hbm.at[sidx.at[i + NB]], bufs.at[b],
                                     gsem.at[b])
                return 0

            lax.fori_loop(0, ngroups, group, 0)

            if h == 1:
                for u in range(UNR - NB):
                    b = u % NB
                    i = h * HALF + (HALF - UNR) + u
                    pltpu.make_async_copy(h_hbm.at[sidx.at[i]], bufs.at[b],
                                          gsem.at[b]).wait()
                    pltpu.sync_copy(bufs.at[b],
                                    acc.at[didx.at[HALF - UNR + u]], add=True)
                    pltpu.async_copy(h_hbm.at[sidx.at[i + NB]], bufs.at[b],
                                     gsem.at[b])

        for b in range(NB):
            i = STEPS - NB + b
            pltpu.make_async_copy(h_hbm.at[sidx.at[i]], bufs.at[b],
                                  gsem.at[b]).wait()
            pltpu.sync_copy(bufs.at[b], acc.at[didx.at[HALF - NB + b]],
                            add=True)

        plsc.subcore_barrier()

        obase = cid * NPAD + sid * RPS
        pltpu.sync_copy(acc.at[pl.ds(sid * RPS, RPS)],
                        out_hbm.at[pl.ds(obase, RPS)])

    return _segsum


_segsum_d = _make_segsum(D)
_segsum_c = _make_segsum(CP)

BN = 2000


def _update_body(relu, m_ref, norm_ref, w_ref, b_ref, o_ref):
    m = m_ref[0] + m_ref[1]
    m = m * norm_ref[...]
    acc = jnp.dot(m, w_ref[...], preferred_element_type=jnp.float32)
    acc = acc + b_ref[...]
    if relu:
        acc = jnp.maximum(acc, 0.0)
    o_ref[...] = acc


def _update(m2, norm, W, b, relu):
    H = W.shape[1]
    return pl.pallas_call(
        functools.partial(_update_body, relu),
        grid=(N // BN,),
        in_specs=[
            pl.BlockSpec((2, BN, D), lambda i: (0, i, 0)),
            pl.BlockSpec((BN, 1), lambda i: (i, 0)),
            pl.BlockSpec((D, H), lambda i: (0, 0)),
            pl.BlockSpec((1, H), lambda i: (0, 0)),
        ],
        out_specs=pl.BlockSpec((BN, H), lambda i: (i, 0)),
        out_shape=jax.ShapeDtypeStruct((N, H), jnp.float32),
    )(m2, norm, W, b.reshape(1, H))


def _update_proj_body(m_ref, norm_ref, w_ref, b_ref, wp_ref, o_ref):
    m = (m_ref[0] + m_ref[1]) * norm_ref[...]
    hid = jnp.dot(m, w_ref[...], preferred_element_type=jnp.float32)
    hid = jnp.maximum(hid + b_ref[...], 0.0)
    o_ref[...] = jnp.dot(hid, wp_ref[...], preferred_element_type=jnp.float32)


def _update_proj(m2, norm, W, b, Wp):
    return pl.pallas_call(
        _update_proj_body,
        grid=(N // BN,),
        in_specs=[
            pl.BlockSpec((2, BN, D), lambda i: (0, i, 0)),
            pl.BlockSpec((BN, 1), lambda i: (i, 0)),
            pl.BlockSpec((D, D), lambda i: (0, 0)),
            pl.BlockSpec((1, D), lambda i: (0, 0)),
            pl.BlockSpec((D, CP), lambda i: (0, 0)),
        ],
        out_specs=pl.BlockSpec((BN, CP), lambda i: (i, 0)),
        out_shape=jax.ShapeDtypeStruct((N, CP), jnp.float32),
    )(m2, norm, W, b.reshape(1, D), Wp)


def _final_body(m_ref, norm_ref, b_ref, o_ref):
    m = m_ref[0, :, :C] + m_ref[1, :, :C]
    o_ref[...] = m * norm_ref[...] + b_ref[...]


def _final(m2, norm, b):
    return pl.pallas_call(
        _final_body,
        grid=(N // BN,),
        in_specs=[
            pl.BlockSpec((2, BN, CP), lambda i: (0, i, 0)),
            pl.BlockSpec((BN, 1), lambda i: (i, 0)),
            pl.BlockSpec((1, C), lambda i: (0, 0)),
        ],
        out_specs=pl.BlockSpec((BN, C), lambda i: (i, 0)),
        out_shape=jax.ShapeDtypeStruct((N, C), jnp.float32),
    )(m2, norm, b.reshape(1, C))


@jax.jit
def kernel(x, edge_index, norm, W0, b0, W1, b1, W2, b2):
    pad = jnp.arange(EPAD - E, dtype=jnp.int32)
    src = jnp.concatenate([edge_index[0], pad % N]).reshape(-1, CH)
    dst = jnp.concatenate([edge_index[1], N + pad % (NPAD - N)]).reshape(-1, CH)
    z = jnp.zeros((NPAD, D), jnp.float32)
    zc = jnp.zeros((NPAD, CP), jnp.float32)

    Wp = jnp.pad(W2, ((0, 0), (0, CP - C)))

    m = _segsum_d(x, src, dst, z).reshape(2, NPAD, D)
    h = _update(m, norm, W0, b0, relu=True)
    m = _segsum_d(h, src, dst, z).reshape(2, NPAD, D)
    p = _update_proj(m, norm, W1, b1, Wp)
    m = _segsum_c(p, src, dst, zc).reshape(2, NPAD, CP)
    return _final(m, norm, b2)


# speedup vs baseline: 1.0138x; 1.0029x over previous
import functools

import jax
import jax.numpy as jnp
from jax import lax
from jax.experimental import pallas as pl
from jax.experimental.pallas import tpu as pltpu
from jax.experimental.pallas import tpu_sc as plsc

N = 10000
E = 320000
D = 128
C = 40
CP = 64

NC = 2
NS = 16
NW = NC * NS
CH = 128
STEPS = 80
HALF = STEPS // 2
EPW = STEPS * CH
EPAD = NW * EPW
NB = 2
NPAD = 10240
RPS = NPAD // NS

_mesh = plsc.VectorSubcoreMesh(core_axis_name="c", subcore_axis_name="s")


def _make_segsum(W):
    @functools.partial(
        pl.kernel,
        out_type=jax.ShapeDtypeStruct((NC * NPAD, W), jnp.float32),
        mesh=_mesh,
        compiler_params=pltpu.CompilerParams(use_tc_tiling_on_sc=False),
        scratch_types=[
            pltpu.VMEM((STEPS, CH), jnp.int32),
            pltpu.VMEM((HALF, CH), jnp.int32),
            pltpu.VMEM((NB, CH, W), jnp.float32),
            pltpu.VMEM_SHARED((NPAD, W), jnp.float32),
            pltpu.SemaphoreType.DMA((NB,)),
            pltpu.SemaphoreType.DMA,
        ],
    )
    def _segsum(h_hbm, src_hbm, dst_hbm, z_hbm, out_hbm, sidx, didx, bufs,
                acc, gsem, zsem):
        cid = lax.axis_index("c")
        sid = lax.axis_index("s")
        wid = cid * NS + sid

        zbase = sid * RPS
        pltpu.async_copy(z_hbm.at[pl.ds(zbase, RPS)],
                         acc.at[pl.ds(zbase, RPS)], zsem)

        pltpu.sync_copy(src_hbm.at[pl.ds(wid * STEPS, STEPS)], sidx)

        for b in range(NB):
            pltpu.async_copy(h_hbm.at[sidx.at[b]], bufs.at[b], gsem.at[b])

        pltpu.make_async_copy(z_hbm.at[pl.ds(zbase, RPS)],
                              acc.at[pl.ds(zbase, RPS)], zsem).wait()
        plsc.subcore_barrier()

        for h in range(2):
            pltpu.sync_copy(dst_hbm.at[pl.ds(wid * STEPS + h * HALF, HALF)],
                            didx)
            ngroups = HALF // NB if h == 0 else HALF // NB - 1

            def group(g, _, h=h):
                for b in range(NB):
                    i = h * HALF + g * NB + b
                    pltpu.make_async_copy(h_hbm.at[sidx.at[i]], bufs.at[b],
                                          gsem.at[b]).wait()
                    pltpu.sync_copy(bufs.at[b], acc.at[didx.at[g * NB + b]],
                                    add=True)
                    pltpu.async_copy(h_hbm.at[sidx.at[i + NB]], bufs.at[b],
                                     gsem.at[b])
                return 0

            lax.fori_loop(0, ngroups, group, 0)

        for b in range(NB):
            i = STEPS - NB + b
            pltpu.make_async_copy(h_hbm.at[sidx.at[i]], bufs.at[b],
                                  gsem.at[b]).wait()
            pltpu.sync_copy(bufs.at[b], acc.at[didx.at[HALF - NB + b]],
                            add=True)

        plsc.subcore_barrier()

        obase = cid * NPAD + sid * RPS
        pltpu.sync_copy(acc.at[pl.ds(sid * RPS, RPS)],
                        out_hbm.at[pl.ds(obase, RPS)])

    return _segsum


_segsum_d = _make_segsum(D)
_segsum_c = _make_segsum(CP)

BN = 2000


def _update_body(relu, m_ref, norm_ref, w_ref, b_ref, o_ref):
    m = m_ref[0] + m_ref[1]
    m = m * norm_ref[...]
    acc = jnp.dot(m, w_ref[...], preferred_element_type=jnp.float32)
    acc = acc + b_ref[...]
    if relu:
        acc = jnp.maximum(acc, 0.0)
    o_ref[...] = acc


def _update(m2, norm, W, b, relu):
    H = W.shape[1]
    return pl.pallas_call(
        functools.partial(_update_body, relu),
        grid=(N // BN,),
        in_specs=[
            pl.BlockSpec((2, BN, D), lambda i: (0, i, 0)),
            pl.BlockSpec((BN, 1), lambda i: (i, 0)),
            pl.BlockSpec((D, H), lambda i: (0, 0)),
            pl.BlockSpec((1, H), lambda i: (0, 0)),
        ],
        out_specs=pl.BlockSpec((BN, H), lambda i: (i, 0)),
        out_shape=jax.ShapeDtypeStruct((N, H), jnp.float32),
    )(m2, norm, W, b.reshape(1, H))


def _update_proj_body(m_ref, norm_ref, w_ref, b_ref, wp_ref, o_ref):
    m = (m_ref[0] + m_ref[1]) * norm_ref[...]
    hid = jnp.dot(m, w_ref[...], preferred_element_type=jnp.float32)
    hid = jnp.maximum(hid + b_ref[...], 0.0)
    o_ref[...] = jnp.dot(hid, wp_ref[...], preferred_element_type=jnp.float32)


def _update_proj(m2, norm, W, b, Wp):
    return pl.pallas_call(
        _update_proj_body,
        grid=(N // BN,),
        in_specs=[
            pl.BlockSpec((2, BN, D), lambda i: (0, i, 0)),
            pl.BlockSpec((BN, 1), lambda i: (i, 0)),
            pl.BlockSpec((D, D), lambda i: (0, 0)),
            pl.BlockSpec((1, D), lambda i: (0, 0)),
            pl.BlockSpec((D, CP), lambda i: (0, 0)),
        ],
        out_specs=pl.BlockSpec((BN, CP), lambda i: (i, 0)),
        out_shape=jax.ShapeDtypeStruct((N, CP), jnp.float32),
    )(m2, norm, W, b.reshape(1, D), Wp)


def _final_body(m_ref, norm_ref, b_ref, o_ref):
    m = m_ref[0, :, :C] + m_ref[1, :, :C]
    o_ref[...] = m * norm_ref[...] + b_ref[...]


def _final(m2, norm, b):
    return pl.pallas_call(
        _final_body,
        grid=(N // BN,),
        in_specs=[
            pl.BlockSpec((2, BN, CP), lambda i: (0, i, 0)),
            pl.BlockSpec((BN, 1), lambda i: (i, 0)),
            pl.BlockSpec((1, C), lambda i: (0, 0)),
        ],
        out_specs=pl.BlockSpec((BN, C), lambda i: (i, 0)),
        out_shape=jax.ShapeDtypeStruct((N, C), jnp.float32),
    )(m2, norm, b.reshape(1, C))


@jax.jit
def kernel(x, edge_index, norm, W0, b0, W1, b1, W2, b2):
    pad = jnp.arange(EPAD - E, dtype=jnp.int32)
    src = jnp.concatenate([edge_index[0], pad % N]).reshape(-1, CH)
    dst = jnp.concatenate([edge_index[1], N + pad % (NPAD - N)]).reshape(-1, CH)
    z = jnp.zeros((NPAD, D), jnp.float32)
    zc = jnp.zeros((NPAD, CP), jnp.float32)

    Wp = jnp.pad(W2, ((0, 0), (0, CP - C)))

    m = _segsum_d(x, src, dst, z).reshape(2, NPAD, D)
    h = _update(m, norm, W0, b0, relu=True)
    m = _segsum_d(h, src, dst, z).reshape(2, NPAD, D)
    p = _update_proj(m, norm, W1, b1, Wp)
    m = _segsum_c(p, src, dst, zc).reshape(2, NPAD, CP)
    return _final(m, norm, b2)
